# Initial kernel scaffold; baseline (speedup 1.0000x reference)
import jax
import jax.numpy as jnp
from jax.experimental import pallas as pl


def kernel(x, edge_index, W1, att_src1, att_dst1, b1, W2, att_src2, att_dst2, b2):
    raise NotImplementedError("write your pallas kernel here")


# speedup vs baseline: 1.0501x; 1.0501x over previous
import jax
import jax.numpy as jnp
from jax.experimental import pallas as pl

H = 8


def _lsm_kernel(x_ref, o_ref):
    x = x_ref[...]
    m = jnp.max(x, axis=-1, keepdims=True)
    e = jnp.exp(x - m)
    s = jnp.sum(e, axis=-1, keepdims=True)
    o_ref[...] = (x - m) - jnp.log(s)


def _log_softmax(x):
    return pl.pallas_call(
        _lsm_kernel, out_shape=jax.ShapeDtypeStruct(x.shape, x.dtype))(x)


def _gat_layer_nomax(x, s, d, W, a_src, a_dst, b, n):
    C = W.shape[1] // H
    xp = (x @ W).reshape(n, H, C)
    asrc = jnp.sum(xp * a_src, axis=-1)
    adst = jnp.sum(xp * a_dst, axis=-1)
    e = jax.nn.leaky_relu(asrc[s] + adst[d], negative_slope=0.2)
    w = jnp.exp(e)
    den = jax.ops.segment_sum(w, d, num_segments=n)
    alpha = w / (den[d] + 1e-16)
    out = jax.ops.segment_sum(xp[s] * alpha[:, :, None], d, num_segments=n)
    out = out.mean(axis=1) + b
    return out, alpha


def kernel(x, edge_index, W1, att_src1, att_dst1, b1,
           W2, att_src2, att_dst2, b2):
    n = x.shape[0]
    loops = jnp.arange(n, dtype=edge_index.dtype)
    ei = jnp.concatenate([edge_index, jnp.stack([loops, loops])], axis=1)
    s, d = ei[0], ei[1]
    h, alpha1 = _gat_layer_nomax(x, s, d, W1, att_src1, att_dst1, b1, n)
    h = jax.nn.relu(h)
    out, alpha2 = _gat_layer_nomax(h, s, d, W2, att_src2, att_dst2, b2, n)
    return _log_softmax(out), ei, alpha1, alpha2


# speedup vs baseline: 1.0903x; 1.0383x over previous
import functools

import jax
import jax.numpy as jnp
from jax import lax
from jax.experimental import pallas as pl
from jax.experimental.pallas import tpu as pltpu
from jax.experimental.pallas import tpu_sc as plsc

H = 8
N = 10000
E1 = 330000
N_PAD = 10112
TRASH = N_PAD - N
SUB = 128
BLK = 1024
NSUB = BLK // SUB
NBLK = 11
EC = BLK * NBLK
E_PAD = 32 * EC
ACC = H * N_PAD
ACC_T = ACC // 16

_mesh = plsc.VectorSubcoreMesh(core_axis_name="c", subcore_axis_name="s")


@functools.partial(
    pl.kernel,
    mesh=_mesh,
    compiler_params=pltpu.CompilerParams(needs_layout_passes=False),
    out_type=[
        jax.ShapeDtypeStruct((H * E_PAD,), jnp.float32),
        jax.ShapeDtypeStruct((2 * ACC,), jnp.float32),
    ],
    scratch_types=[
        pltpu.VMEM((EC,), jnp.int32),
        pltpu.VMEM((EC,), jnp.int32),
        pltpu.VMEM((N_PAD,), jnp.float32),
        pltpu.VMEM((N_PAD,), jnp.float32),
        pltpu.VMEM((BLK,), jnp.float32),
        pltpu.VMEM((NSUB, SUB), jnp.int32),
        pltpu.VMEM((ACC_T,), jnp.float32),
        pltpu.VMEM_SHARED((ACC,), jnp.float32),
    ],
)
def _phase_a(as_t, ad_t, s_all_h, d_all_h, w_out, den_out,
             s_all, d_all, as_h, ad_h, wbuf, idxb, zbuf, acc):
    c = lax.axis_index("c")
    sid = lax.axis_index("s")
    wid = sid * 2 + c

    pltpu.sync_copy(s_all_h.at[pl.ds(wid * EC, EC)], s_all)
    pltpu.sync_copy(d_all_h.at[pl.ds(wid * EC, EC)], d_all)

    def zloop(i, carry):
        zbuf[pl.ds(i * 16, 16)] = jnp.zeros((16,), jnp.float32)
        return carry
    lax.fori_loop(0, ACC_T // 16, zloop, 0)
    pltpu.sync_copy(zbuf, acc.at[pl.ds(sid * ACC_T, ACC_T)])
    plsc.subcore_barrier()

    for h in range(H):
        pltpu.sync_copy(as_t.at[pl.ds(h * N_PAD, N_PAD)], as_h)
        pltpu.sync_copy(ad_t.at[pl.ds(h * N_PAD, N_PAD)], ad_h)

        def blk(b, carry):
            base = b * BLK
            for j in range(NSUB):
                def chunk(k, cc):
                    off = base + j * SUB + k * 16
                    s_vec = s_all[pl.ds(off, 16)]
                    d_vec = d_all[pl.ds(off, 16)]
                    sv = plsc.load_gather(as_h, [s_vec])
                    dv = plsc.load_gather(ad_h, [d_vec])
                    v = sv + dv
                    v = jnp.maximum(v, v * jnp.float32(0.2))
                    wbuf[pl.ds(j * SUB + k * 16, 16)] = jnp.exp(v)
                    idxb[j, pl.ds(k * 16, 16)] = d_vec + h * N_PAD
                    return cc
                lax.fori_loop(0, SUB // 16, chunk, 0)
            pltpu.sync_copy(
                wbuf, w_out.at[pl.ds(h * E_PAD + wid * EC + base, BLK)])
            for j in range(NSUB):
                pltpu.sync_copy(wbuf.at[pl.ds(j * SUB, SUB)],
                                acc.at[idxb.at[j]], add=True)
            return carry
        lax.fori_loop(0, NBLK, blk, 0)

    plsc.subcore_barrier()
    pltpu.sync_copy(acc.at[pl.ds(sid * ACC_T, ACC_T)], zbuf)
    pltpu.sync_copy(zbuf, den_out.at[pl.ds(c * ACC + sid * ACC_T, ACC_T)])


def _lsm_kernel(x_ref, o_ref):
    x = x_ref[...]
    m = jnp.max(x, axis=-1, keepdims=True)
    e = jnp.exp(x - m)
    s = jnp.sum(e, axis=-1, keepdims=True)
    o_ref[...] = (x - m) - jnp.log(s)


def _log_softmax(x):
    return pl.pallas_call(
        _lsm_kernel, out_shape=jax.ShapeDtypeStruct(x.shape, x.dtype))(x)


def _gat_layer(x, s, d, s_pad, d_pad, W, a_src, a_dst, b, n):
    C = W.shape[1] // H
    xp = (x @ W).reshape(n, H, C)
    asrc = jnp.sum(xp * a_src, axis=-1)
    adst = jnp.sum(xp * a_dst, axis=-1)
    as_t = jnp.pad(asrc, ((0, TRASH), (0, 0))).T.reshape(-1)
    ad_t = jnp.pad(adst, ((0, TRASH), (0, 0))).T.reshape(-1)
    w_hm, den2 = _phase_a(as_t, ad_t, s_pad, d_pad)
    den2 = den2.reshape(2, H, N_PAD)
    den = (den2[0] + den2[1])[:, :n]
    w = w_hm.reshape(H, E_PAD)[:, :E1]
    alpha = (w / (den[:, d] + 1e-16)).T
    out = jax.ops.segment_sum(xp[s] * alpha[:, :, None], d, num_segments=n)
    out = out.mean(axis=1) + b
    return out, alpha


def kernel(x, edge_index, W1, att_src1, att_dst1, b1,
           W2, att_src2, att_dst2, b2):
    n = x.shape[0]
    loops = jnp.arange(n, dtype=edge_index.dtype)
    ei = jnp.concatenate([edge_index, jnp.stack([loops, loops])], axis=1)
    s, d = ei[0], ei[1]
    padn = E_PAD - E1
    spread = (jnp.arange(padn, dtype=jnp.int32) % TRASH)
    s_pad = jnp.concatenate([s, spread])
    d_pad = jnp.concatenate([d, N + spread])
    h, alpha1 = _gat_layer(x, s, d, s_pad, d_pad, W1, att_src1, att_dst1,
                           b1, n)
    h = jax.nn.relu(h)
    out, alpha2 = _gat_layer(h, s, d, s_pad, d_pad, W2, att_src2, att_dst2,
                             b2, n)
    return _log_softmax(out), ei, alpha1, alpha2


# speedup vs baseline: 18.8543x; 17.2931x over previous
import functools

import jax
import jax.numpy as jnp
from jax import lax
from jax.experimental import pallas as pl
from jax.experimental.pallas import tpu as pltpu
from jax.experimental.pallas import tpu_sc as plsc

H = 8
N = 10000
E1 = 330000
N_PAD = 10112
TRASH = N_PAD - N
SUB = 128
BLK = 1024
NSUB = BLK // SUB
NBLK = 11
EC = BLK * NBLK
E_PAD = 32 * EC
ACC = H * N_PAD
ACC_T = ACC // 16
PB = 128
ECB = E_PAD // 16
NPB = ECB // PB
ROWS_T = N_PAD // 16

_mesh = plsc.VectorSubcoreMesh(core_axis_name="c", subcore_axis_name="s")
_params = pltpu.CompilerParams(needs_layout_passes=False)


@functools.partial(
    pl.kernel,
    mesh=_mesh,
    compiler_params=_params,
    out_type=[
        jax.ShapeDtypeStruct((2 * ACC,), jnp.float32),
    ],
    scratch_types=[
        pltpu.VMEM((EC,), jnp.int32),
        pltpu.VMEM((EC,), jnp.int32),
        pltpu.VMEM((N_PAD,), jnp.float32),
        pltpu.VMEM((N_PAD,), jnp.float32),
        pltpu.VMEM((BLK,), jnp.float32),
        pltpu.VMEM((NSUB, SUB), jnp.int32),
        pltpu.VMEM((ACC_T,), jnp.float32),
        pltpu.VMEM_SHARED((ACC,), jnp.float32),
    ],
)
def _phase_a(as_t, ad_t, s_all_h, d_all_h, den_out,
             s_all, d_all, as_h, ad_h, wbuf, idxb, zbuf, acc):
    c = lax.axis_index("c")
    sid = lax.axis_index("s")
    wid = sid * 2 + c

    pltpu.sync_copy(s_all_h.at[pl.ds(wid * EC, EC)], s_all)
    pltpu.sync_copy(d_all_h.at[pl.ds(wid * EC, EC)], d_all)

    def zloop(i, carry):
        zbuf[pl.ds(i * 16, 16)] = jnp.zeros((16,), jnp.float32)
        return carry
    lax.fori_loop(0, ACC_T // 16, zloop, 0)
    pltpu.sync_copy(zbuf, acc.at[pl.ds(sid * ACC_T, ACC_T)])
    plsc.subcore_barrier()

    for h in range(H):
        pltpu.sync_copy(as_t.at[pl.ds(h * N_PAD, N_PAD)], as_h)
        pltpu.sync_copy(ad_t.at[pl.ds(h * N_PAD, N_PAD)], ad_h)

        def blk(b, carry):
            base = b * BLK
            for j in range(NSUB):
                def chunk(k, cc):
                    off = base + j * SUB + k * 16
                    s_vec = s_all[pl.ds(off, 16)]
                    d_vec = d_all[pl.ds(off, 16)]
                    sv = plsc.load_gather(as_h, [s_vec])
                    dv = plsc.load_gather(ad_h, [d_vec])
                    v = sv + dv
                    v = jnp.maximum(v, v * jnp.float32(0.2))
                    wbuf[pl.ds(j * SUB + k * 16, 16)] = jnp.exp(v)
                    idxb[j, pl.ds(k * 16, 16)] = d_vec + h * N_PAD
                    return cc
                lax.fori_loop(0, SUB // 16, chunk, 0)
            for j in range(NSUB):
                pltpu.sync_copy(wbuf.at[pl.ds(j * SUB, SUB)],
                                acc.at[idxb.at[j]], add=True)
            return carry
        lax.fori_loop(0, NBLK, blk, 0)

    plsc.subcore_barrier()
    pltpu.sync_copy(acc.at[pl.ds(sid * ACC_T, ACC_T)], zbuf)
    pltpu.sync_copy(zbuf, den_out.at[pl.ds(c * ACC + sid * ACC_T, ACC_T)])


TBL = 3 * ACC
TBL_T = TBL // 16
STG = 1024


@functools.partial(
    pl.kernel,
    mesh=_mesh,
    compiler_params=_params,
    out_type=[
        jax.ShapeDtypeStruct((H * E_PAD,), jnp.float32),
        jax.ShapeDtypeStruct((4 * N_PAD, 128), jnp.float32),
    ],
    scratch_types=[
        pltpu.VMEM((STG,), jnp.float32),
        pltpu.VMEM((PB, 128), jnp.float32),
        pltpu.VMEM((PB,), jnp.int32),
        pltpu.VMEM((PB,), jnp.int32),
        pltpu.VMEM((6, PB), jnp.int32),
        pltpu.VMEM((6, PB), jnp.float32),
        pltpu.VMEM((1, PB), jnp.int32),
        pltpu.VMEM((1, PB), jnp.int32),
        pltpu.VMEM((PB + 16,), jnp.float32),
        pltpu.VMEM((PB + 16,), jnp.float32),
        pltpu.VMEM_SHARED((TBL,), jnp.float32),
        pltpu.VMEM_SHARED((N_PAD, 128), jnp.float32),
    ],
)
def _phase_b(xp_all, tbl_h, s_all_h, d_all_h,
             alpha_out, u_out,
             stage, xpb, s_b, d_b, idx6, val6, sidx, didx, al0, al1,
             tbl, acc):
    c = lax.axis_index("c")
    sid = lax.axis_index("s")
    wid = sid * 2 + c

    for k in range(TBL_T // STG + 1):
        ln = STG if (k + 1) * STG <= TBL_T else TBL_T - k * STG
        if ln <= 0:
            break
        off = sid * TBL_T + k * STG
        pltpu.sync_copy(tbl_h.at[pl.ds(off, ln)], stage.at[pl.ds(0, ln)])
        pltpu.sync_copy(stage.at[pl.ds(0, ln)], tbl.at[pl.ds(off, ln)])

    for pp in range(2):
        p = 2 * c + pp
        h0 = 2 * p

        def z16(i, carry):
            xpb[i // 8, pl.ds((i % 8) * 16, 16)] = jnp.zeros((16,),
                                                             jnp.float32)
            return carry
        lax.fori_loop(0, 64 * 8, z16, 0)
        for k in range(9):
            pltpu.sync_copy(xpb.at[pl.ds(0, 64)],
                            acc.at[pl.ds(sid * ROWS_T + k * 64, 64)])
        pltpu.sync_copy(xpb.at[pl.ds(0, 56)],
                        acc.at[pl.ds(sid * ROWS_T + 576, 56)])
        plsc.subcore_barrier()

        def batch(bt, carry):
            goff = sid * ECB + bt * PB
            pltpu.sync_copy(s_all_h.at[pl.ds(goff, PB)], s_b)
            pltpu.sync_copy(d_all_h.at[pl.ds(goff, PB)], d_b)

            def bld(k, cc):
                sl = pl.ds(k * 16, 16)
                s_vec = s_b[sl]
                d_vec = d_b[sl]
                idx6[0, sl] = s_vec + h0 * N_PAD
                idx6[1, sl] = s_vec + (h0 + 1) * N_PAD
                idx6[2, sl] = d_vec + (ACC + h0 * N_PAD)
                idx6[3, sl] = d_vec + (ACC + (h0 + 1) * N_PAD)
                idx6[4, sl] = d_vec + (2 * ACC + h0 * N_PAD)
                idx6[5, sl] = d_vec + (2 * ACC + (h0 + 1) * N_PAD)
                sidx[0, sl] = s_vec + p * N_PAD
                didx[0, sl] = d_vec
                return cc
            lax.fori_loop(0, PB // 16, bld, 0)

            for r in range(6):
                pltpu.sync_copy(tbl.at[idx6.at[r]], val6.at[r])
            pltpu.sync_copy(xp_all.at[sidx.at[0]], xpb)

            def chunk(k, cc):
                sl = pl.ds(k * 16, 16)
                v0 = val6[0, sl] + val6[2, sl]
                v0 = jnp.maximum(v0, v0 * jnp.float32(0.2))
                v1 = val6[1, sl] + val6[3, sl]
                v1 = jnp.maximum(v1, v1 * jnp.float32(0.2))
                al0[sl] = jnp.exp(v0) / val6[4, sl]
                al1[sl] = jnp.exp(v1) / val6[5, sl]
                return cc
            lax.fori_loop(0, PB // 16, chunk, 0)

            pltpu.sync_copy(
                al0.at[pl.ds(0, PB)],
                alpha_out.at[pl.ds(h0 * E_PAD + goff, PB)])
            pltpu.sync_copy(
                al1.at[pl.ds(0, PB)],
                alpha_out.at[pl.ds((h0 + 1) * E_PAD + goff, PB)])

            def edge(e2, cc):
                va = jnp.full((16,), al0[pl.ds(e2, 16)][0], jnp.float32)
                vb = jnp.full((16,), al1[pl.ds(e2, 16)][0], jnp.float32)
                for cc4 in range(4):
                    sl = pl.ds(cc4 * 16, 16)
                    xpb[e2, sl] = xpb[e2, sl] * va
                for cc4 in range(4, 8):
                    sl = pl.ds(cc4 * 16, 16)
                    xpb[e2, sl] = xpb[e2, sl] * vb
                return cc
            lax.fori_loop(0, PB, edge, 0)

            pltpu.sync_copy(xpb, acc.at[didx.at[0]], add=True)
            return carry
        lax.fori_loop(0, NPB, batch, 0)
        plsc.subcore_barrier()

        for k in range(9):
            pltpu.sync_copy(acc.at[pl.ds(sid * ROWS_T + k * 64, 64)],
                            xpb.at[pl.ds(0, 64)])
            pltpu.sync_copy(
                xpb.at[pl.ds(0, 64)],
                u_out.at[pl.ds(p * N_PAD + sid * ROWS_T + k * 64, 64)])
        pltpu.sync_copy(acc.at[pl.ds(sid * ROWS_T + 576, 56)],
                        xpb.at[pl.ds(0, 56)])
        pltpu.sync_copy(
            xpb.at[pl.ds(0, 56)],
            u_out.at[pl.ds(p * N_PAD + sid * ROWS_T + 576, 56)])
        plsc.subcore_barrier()


def _lsm_kernel(x_ref, o_ref):
    x = x_ref[...]
    m = jnp.max(x, axis=-1, keepdims=True)
    e = jnp.exp(x - m)
    s = jnp.sum(e, axis=-1, keepdims=True)
    o_ref[...] = (x - m) - jnp.log(s)


def _log_softmax(x):
    return pl.pallas_call(
        _lsm_kernel, out_shape=jax.ShapeDtypeStruct(x.shape, x.dtype))(x)


def _gat_layer(x, s_pad, d_pad, W, a_src, a_dst, b, n):
    C = W.shape[1] // H
    xp = (x @ W).reshape(n, H, C)
    asrc = jnp.sum(xp * a_src, axis=-1)
    adst = jnp.sum(xp * a_dst, axis=-1)
    as_t = jnp.pad(asrc, ((0, TRASH), (0, 0))).T.reshape(-1)
    ad_t = jnp.pad(adst, ((0, TRASH), (0, 0))).T.reshape(-1)
    [den2] = _phase_a(as_t, ad_t, s_pad, d_pad)
    den_sum = den2[:ACC] + den2[ACC:] + 1e-16
    tbl_h = jnp.concatenate([as_t, ad_t, den_sum])
    xp_all = (jnp.pad(xp, ((0, TRASH), (0, 0), (0, 0)))
              .reshape(N_PAD, 4, 128).transpose(1, 0, 2)
              .reshape(4 * N_PAD, 128))
    alpha_hm, u = _phase_b(xp_all, tbl_h, s_pad, d_pad)
    alpha = alpha_hm.reshape(H, E_PAD)[:, :E1].T
    u4 = (u.reshape(4, N_PAD, 2, C)[:, :n]
          .transpose(1, 0, 2, 3).reshape(n, H, C))
    out = u4.mean(axis=1) + b
    return out, alpha


def kernel(x, edge_index, W1, att_src1, att_dst1, b1,
           W2, att_src2, att_dst2, b2):
    n = x.shape[0]
    loops = jnp.arange(n, dtype=edge_index.dtype)
    ei = jnp.concatenate([edge_index, jnp.stack([loops, loops])], axis=1)
    s, d = ei[0], ei[1]
    padn = E_PAD - E1
    spread = (jnp.arange(padn, dtype=jnp.int32) % TRASH)
    s_pad = jnp.concatenate([s, spread])
    d_pad = jnp.concatenate([d, N + spread])
    h, alpha1 = _gat_layer(x, s_pad, d_pad, W1, att_src1, att_dst1, b1, n)
    h = jax.nn.relu(h)
    out, alpha2 = _gat_layer(h, s_pad, d_pad, W2, att_src2, att_dst2, b2, n)
    return _log_softmax(out), ei, alpha1, alpha2


# speedup vs baseline: 20.4646x; 1.0854x over previous
import functools

import jax
import jax.numpy as jnp
from jax import lax
from jax.experimental import pallas as pl
from jax.experimental.pallas import tpu as pltpu
from jax.experimental.pallas import tpu_sc as plsc

H = 8
N = 10000
E1 = 330000
N_PAD = 10112
TRASH = N_PAD - N
SUB = 128
BLK = 1024
NSUB = BLK // SUB
NBLK = 11
EC = BLK * NBLK
E_PAD = 32 * EC
ACC = H * N_PAD
ACC_T = ACC // 16
PB = 128
ECB = E_PAD // 16
NPB = ECB // PB
ROWS_T = N_PAD // 16

_mesh = plsc.VectorSubcoreMesh(core_axis_name="c", subcore_axis_name="s")
_params = pltpu.CompilerParams(needs_layout_passes=False)


@functools.partial(
    pl.kernel,
    mesh=_mesh,
    compiler_params=_params,
    out_type=[
        jax.ShapeDtypeStruct((2 * ACC,), jnp.float32),
    ],
    scratch_types=[
        pltpu.VMEM((EC,), jnp.int32),
        pltpu.VMEM((EC,), jnp.int32),
        pltpu.VMEM((N_PAD,), jnp.float32),
        pltpu.VMEM((N_PAD,), jnp.float32),
        pltpu.VMEM((BLK,), jnp.float32),
        pltpu.VMEM((NSUB, SUB), jnp.int32),
        pltpu.VMEM((ACC_T,), jnp.float32),
        pltpu.VMEM_SHARED((ACC,), jnp.float32),
    ],
)
def _phase_a(as_t, ad_t, s_all_h, d_all_h, den_out,
             s_all, d_all, as_h, ad_h, wbuf, idxb, zbuf, acc):
    c = lax.axis_index("c")
    sid = lax.axis_index("s")
    wid = sid * 2 + c

    pltpu.sync_copy(s_all_h.at[pl.ds(wid * EC, EC)], s_all)
    pltpu.sync_copy(d_all_h.at[pl.ds(wid * EC, EC)], d_all)

    def zloop(i, carry):
        zbuf[pl.ds(i * 16, 16)] = jnp.zeros((16,), jnp.float32)
        return carry
    lax.fori_loop(0, ACC_T // 16, zloop, 0)
    pltpu.sync_copy(zbuf, acc.at[pl.ds(sid * ACC_T, ACC_T)])
    plsc.subcore_barrier()

    for h in range(H):
        pltpu.sync_copy(as_t.at[pl.ds(h * N_PAD, N_PAD)], as_h)
        pltpu.sync_copy(ad_t.at[pl.ds(h * N_PAD, N_PAD)], ad_h)

        def blk(b, carry):
            base = b * BLK
            for j in range(NSUB):
                def chunk(k, cc):
                    off = base + j * SUB + k * 16
                    s_vec = s_all[pl.ds(off, 16)]
                    d_vec = d_all[pl.ds(off, 16)]
                    sv = plsc.load_gather(as_h, [s_vec])
                    dv = plsc.load_gather(ad_h, [d_vec])
                    v = sv + dv
                    v = jnp.maximum(v, v * jnp.float32(0.2))
                    wbuf[pl.ds(j * SUB + k * 16, 16)] = jnp.exp(v)
                    idxb[j, pl.ds(k * 16, 16)] = d_vec + h * N_PAD
                    return cc
                lax.fori_loop(0, SUB // 16, chunk, 0)
            for j in range(NSUB):
                pltpu.sync_copy(wbuf.at[pl.ds(j * SUB, SUB)],
                                acc.at[idxb.at[j]], add=True)
            return carry
        lax.fori_loop(0, NBLK, blk, 0)

    plsc.subcore_barrier()
    pltpu.sync_copy(acc.at[pl.ds(sid * ACC_T, ACC_T)], zbuf)
    pltpu.sync_copy(zbuf, den_out.at[pl.ds(c * ACC + sid * ACC_T, ACC_T)])


TBL = 3 * ACC
TBL_T = TBL // 16
STG = 1024


@functools.partial(
    pl.kernel,
    mesh=_mesh,
    compiler_params=_params,
    out_type=[
        jax.ShapeDtypeStruct((H * E_PAD,), jnp.float32),
        jax.ShapeDtypeStruct((4 * N_PAD, 128), jnp.float32),
    ],
    scratch_types=[
        pltpu.VMEM((STG,), jnp.float32),
        pltpu.VMEM((PB, 128), jnp.float32),
        pltpu.VMEM((PB,), jnp.int32),
        pltpu.VMEM((PB,), jnp.int32),
        pltpu.VMEM((6 * PB,), jnp.int32),
        pltpu.VMEM((6 * PB,), jnp.float32),
        pltpu.VMEM((1, PB), jnp.int32),
        pltpu.VMEM((1, PB), jnp.int32),
        pltpu.VMEM((PB + 16,), jnp.float32),
        pltpu.VMEM((PB + 16,), jnp.float32),
        pltpu.VMEM_SHARED((TBL,), jnp.float32),
        pltpu.VMEM_SHARED((N_PAD, 128), jnp.float32),
    ],
)
def _phase_b(xp_all, tbl_h, s_all_h, d_all_h,
             alpha_out, u_out,
             stage, xpb, s_b, d_b, idx6, val6, sidx, didx, al0, al1,
             tbl, acc):
    c = lax.axis_index("c")
    sid = lax.axis_index("s")
    wid = sid * 2 + c

    for k in range(TBL_T // STG + 1):
        ln = STG if (k + 1) * STG <= TBL_T else TBL_T - k * STG
        if ln <= 0:
            break
        off = sid * TBL_T + k * STG
        pltpu.sync_copy(tbl_h.at[pl.ds(off, ln)], stage.at[pl.ds(0, ln)])
        pltpu.sync_copy(stage.at[pl.ds(0, ln)], tbl.at[pl.ds(off, ln)])

    for pp in range(2):
        p = 2 * c + pp
        h0 = 2 * p

        def z16(i, carry):
            xpb[i // 8, pl.ds((i % 8) * 16, 16)] = jnp.zeros((16,),
                                                             jnp.float32)
            return carry
        lax.fori_loop(0, 64 * 8, z16, 0)
        for k in range(9):
            pltpu.sync_copy(xpb.at[pl.ds(0, 64)],
                            acc.at[pl.ds(sid * ROWS_T + k * 64, 64)])
        pltpu.sync_copy(xpb.at[pl.ds(0, 56)],
                        acc.at[pl.ds(sid * ROWS_T + 576, 56)])
        plsc.subcore_barrier()

        def batch(bt, carry):
            goff = sid * ECB + bt * PB
            pltpu.sync_copy(s_all_h.at[pl.ds(goff, PB)], s_b)
            pltpu.sync_copy(d_all_h.at[pl.ds(goff, PB)], d_b)

            def bld(k, cc):
                sl = pl.ds(k * 16, 16)
                s_vec = s_b[sl]
                d_vec = d_b[sl]
                o = k * 16
                idx6[pl.ds(o, 16)] = s_vec + h0 * N_PAD
                idx6[pl.ds(PB + o, 16)] = s_vec + (h0 + 1) * N_PAD
                idx6[pl.ds(2 * PB + o, 16)] = d_vec + (ACC + h0 * N_PAD)
                idx6[pl.ds(3 * PB + o, 16)] = d_vec + (ACC + (h0 + 1) * N_PAD)
                idx6[pl.ds(4 * PB + o, 16)] = d_vec + (2 * ACC + h0 * N_PAD)
                idx6[pl.ds(5 * PB + o, 16)] = (d_vec
                                               + (2 * ACC + (h0 + 1) * N_PAD))
                sidx[0, sl] = s_vec + p * N_PAD
                didx[0, sl] = d_vec
                return cc
            lax.fori_loop(0, PB // 16, bld, 0)

            pltpu.sync_copy(tbl.at[idx6], val6)
            pltpu.sync_copy(xp_all.at[sidx.at[0]], xpb)

            def chunk(k, cc):
                sl = pl.ds(k * 16, 16)
                o = k * 16
                v0 = val6[pl.ds(o, 16)] + val6[pl.ds(2 * PB + o, 16)]
                v0 = jnp.maximum(v0, v0 * jnp.float32(0.2))
                v1 = val6[pl.ds(PB + o, 16)] + val6[pl.ds(3 * PB + o, 16)]
                v1 = jnp.maximum(v1, v1 * jnp.float32(0.2))
                al0[sl] = jnp.exp(v0) / val6[pl.ds(4 * PB + o, 16)]
                al1[sl] = jnp.exp(v1) / val6[pl.ds(5 * PB + o, 16)]
                return cc
            lax.fori_loop(0, PB // 16, chunk, 0)

            pltpu.sync_copy(
                al0.at[pl.ds(0, PB)],
                alpha_out.at[pl.ds(h0 * E_PAD + goff, PB)])
            pltpu.sync_copy(
                al1.at[pl.ds(0, PB)],
                alpha_out.at[pl.ds((h0 + 1) * E_PAD + goff, PB)])

            def edge(e2, cc):
                va = jnp.full((16,), al0[pl.ds(e2, 16)][0], jnp.float32)
                vb = jnp.full((16,), al1[pl.ds(e2, 16)][0], jnp.float32)
                for cc4 in range(4):
                    sl = pl.ds(cc4 * 16, 16)
                    xpb[e2, sl] = xpb[e2, sl] * va
                for cc4 in range(4, 8):
                    sl = pl.ds(cc4 * 16, 16)
                    xpb[e2, sl] = xpb[e2, sl] * vb
                return cc
            lax.fori_loop(0, PB, edge, 0)

            pltpu.sync_copy(xpb, acc.at[didx.at[0]], add=True)
            return carry
        lax.fori_loop(0, NPB, batch, 0)
        plsc.subcore_barrier()

        for k in range(9):
            pltpu.sync_copy(acc.at[pl.ds(sid * ROWS_T + k * 64, 64)],
                            xpb.at[pl.ds(0, 64)])
            pltpu.sync_copy(
                xpb.at[pl.ds(0, 64)],
                u_out.at[pl.ds(p * N_PAD + sid * ROWS_T + k * 64, 64)])
        pltpu.sync_copy(acc.at[pl.ds(sid * ROWS_T + 576, 56)],
                        xpb.at[pl.ds(0, 56)])
        pltpu.sync_copy(
            xpb.at[pl.ds(0, 56)],
            u_out.at[pl.ds(p * N_PAD + sid * ROWS_T + 576, 56)])
        plsc.subcore_barrier()


def _lsm_kernel(x_ref, o_ref):
    x = x_ref[...]
    m = jnp.max(x, axis=-1, keepdims=True)
    e = jnp.exp(x - m)
    s = jnp.sum(e, axis=-1, keepdims=True)
    o_ref[...] = (x - m) - jnp.log(s)


def _log_softmax(x):
    return pl.pallas_call(
        _lsm_kernel, out_shape=jax.ShapeDtypeStruct(x.shape, x.dtype))(x)


def _gat_layer(x, s_pad, d_pad, W, a_src, a_dst, b, n):
    C = W.shape[1] // H
    xp = (x @ W).reshape(n, H, C)
    asrc = jnp.sum(xp * a_src, axis=-1)
    adst = jnp.sum(xp * a_dst, axis=-1)
    as_t = jnp.pad(asrc, ((0, TRASH), (0, 0))).T.reshape(-1)
    ad_t = jnp.pad(adst, ((0, TRASH), (0, 0))).T.reshape(-1)
    [den2] = _phase_a(as_t, ad_t, s_pad, d_pad)
    den_sum = den2[:ACC] + den2[ACC:] + 1e-16
    tbl_h = jnp.concatenate([as_t, ad_t, den_sum])
    xp_all = (jnp.pad(xp, ((0, TRASH), (0, 0), (0, 0)))
              .reshape(N_PAD, 4, 128).transpose(1, 0, 2)
              .reshape(4 * N_PAD, 128))
    alpha_hm, u = _phase_b(xp_all, tbl_h, s_pad, d_pad)
    alpha = alpha_hm.reshape(H, E_PAD)[:, :E1].T
    u4 = (u.reshape(4, N_PAD, 2, C)[:, :n]
          .transpose(1, 0, 2, 3).reshape(n, H, C))
    out = u4.mean(axis=1) + b
    return out, alpha


def kernel(x, edge_index, W1, att_src1, att_dst1, b1,
           W2, att_src2, att_dst2, b2):
    n = x.shape[0]
    loops = jnp.arange(n, dtype=edge_index.dtype)
    ei = jnp.concatenate([edge_index, jnp.stack([loops, loops])], axis=1)
    s, d = ei[0], ei[1]
    padn = E_PAD - E1
    spread = (jnp.arange(padn, dtype=jnp.int32) % TRASH)
    s_pad = jnp.concatenate([s, spread])
    d_pad = jnp.concatenate([d, N + spread])
    h, alpha1 = _gat_layer(x, s_pad, d_pad, W1, att_src1, att_dst1, b1, n)
    h = jax.nn.relu(h)
    out, alpha2 = _gat_layer(h, s_pad, d_pad, W2, att_src2, att_dst2, b2, n)
    return _log_softmax(out), ei, alpha1, alpha2
